# Initial kernel scaffold; baseline (speedup 1.0000x reference)
import jax
import jax.numpy as jnp
from jax.experimental import pallas as pl


def kernel(gt_src_corr_indices, gt_tgt_corr_indices, gt_corr_overlaps):
    raise NotImplementedError("write your pallas kernel here")


# speedup vs baseline: 3.0194x; 3.0194x over previous
import functools

import jax
import jax.numpy as jnp
from jax import lax
from jax.experimental import pallas as pl
from jax.experimental.pallas import tpu as pltpu
from jax.experimental.pallas import tpu_sc as plsc

N = 100000
K = 256
L = 16
NUM_TILES = 16
PER_TILE = 6256
VREGS = PER_TILE // L
PAD_N = NUM_TILES * PER_TILE
CAP = 4096
NEG_KEY = -2147483648
PAD_IDX = 2147483647


def _body(pert_hbm, src_hbm, tgt_hbm, ovl_hbm,
          out_src, out_tgt, out_ovl,
          buf, keys, hist, suf, candk, candi, gk, gi, gridbuf,
          selbuf, selrows, selidx, scal8, sc16x8, gsrc, gtgt, govl,
          sh_hist, sh_scal, sh_gk, sh_gi, sh_sel, sem):
    c = lax.axis_index("c")
    s = lax.axis_index("s")
    iota = lax.iota(jnp.int32, L)
    ones = jnp.ones((L,), jnp.int32)
    zeros = jnp.zeros((L,), jnp.int32)

    base = s * PER_TILE
    pltpu.sync_copy(pert_hbm.at[pl.ds(base, PER_TILE)], buf)

    def _zero_hist(i, _):
        hist[pl.ds(i * L, L)] = zeros
        return 0
    lax.fori_loop(0, 256, _zero_hist, 0)

    lane_a = iota * 256 + 128

    def _pass_a(j, _):
        raw = buf[pl.ds(j * L, L)]
        keyv = raw ^ ((raw >> 31) & 0x7FFFFFFF)
        keys[pl.ds(j * L, L)] = keyv
        plsc.addupdate_scatter(hist, [(keyv >> 24) + lane_a], ones)
        return 0
    lax.fori_loop(0, VREGS, _pass_a, 0)

    def _merge_hist(tot_ref):
        def _red(g, _):
            acc = hist[pl.ds(g * L, L)]
            for lane in range(1, L):
                acc = acc + hist[pl.ds(lane * 256 + g * L, L)]
            tot_ref[pl.ds(g * L, L)] = acc
            return 0
        lax.fori_loop(0, L, _red, 0)
        pltpu.sync_copy(tot_ref, sh_hist.at[s])
        plsc.subcore_barrier()
        pltpu.sync_copy(sh_hist, gridbuf)
        plsc.subcore_barrier()
        def _sum(g, _):
            acc = gridbuf[0, pl.ds(g * L, L)]
            for t in range(1, NUM_TILES):
                acc = acc + gridbuf[t, pl.ds(g * L, L)]
            tot_ref[pl.ds(g * L, L)] = acc
            return 0
        lax.fori_loop(0, L, _sum, 0)

    def _suffix_select(tot_ref, need):
        carry = zeros
        accq = zeros
        for g in range(L - 1, -1, -1):
            tv = tot_ref[pl.ds(g * L, L)]
            sincl = lax.rev(plsc.cumsum(lax.rev(tv, (0,))), (0,)) + carry
            suf[pl.ds(g * L, L)] = sincl
            carry = carry + jnp.full((L,), jnp.sum(tv), jnp.int32)
            accq = accq + plsc.all_reduce_population_count(sincl >= need)
        bstar = jnp.max(accq) - 1
        nxt = jnp.minimum(bstar + 1, 255)
        suf_next_v = plsc.load_gather(suf, [jnp.full((L,), nxt, jnp.int32)])
        suf_next = jnp.where(bstar >= 255, 0, jnp.max(suf_next_v))
        return bstar, suf_next

    totA = selbuf
    _merge_hist(totA)
    bstar, sufA_next = _suffix_select(totA, K)
    need_b = K - sufA_next
    b8 = bstar - 128

    lax.fori_loop(0, 256, _zero_hist, 0)
    lane_b = iota * 256

    def _pass_b(j, _):
        keyv = keys[pl.ds(j * L, L)]
        m = (keyv >> 24) == b8
        plsc.addupdate_scatter(hist, [((keyv >> 16) & 0xFF) + lane_b], ones,
                               mask=m)
        return 0
    lax.fori_loop(0, VREGS, _pass_b, 0)

    _merge_hist(totA)
    cstar, _ = _suffix_select(totA, need_b)
    thresh = (b8 << 24) + (cstar << 16)

    def _compact(j, off):
        keyv = keys[pl.ds(j * L, L)]
        m = jnp.logical_and(keyv >= thresh, off < CAP)
        cnt = plsc.all_reduce_population_count(m)
        gidx = jnp.full((L,), base + j * L, jnp.int32) + iota
        plsc.store_compressed(candk.at[pl.ds(off, L)], keyv, mask=m)
        plsc.store_compressed(candi.at[pl.ds(off, L)], gidx, mask=m)
        return off + jnp.max(cnt)
    off = lax.fori_loop(0, VREGS, _compact, jnp.int32(0))

    candk[pl.ds(off, L)] = jnp.full((L,), NEG_KEY, jnp.int32)
    candi[pl.ds(off, L)] = jnp.full((L,), PAD_IDX, jnp.int32)
    nv = (off + L - 1) // L

    scal8[...] = jnp.full((L,), nv, jnp.int32)
    pltpu.sync_copy(scal8.at[pl.ds(0, 8)], sh_scal.at[s])
    plsc.subcore_barrier()
    pltpu.sync_copy(sh_scal, sc16x8)
    nvs = plsc.load_gather(sc16x8, [iota, zeros])
    incl = plsc.cumsum(nvs)
    my_slot = jnp.max(jnp.where(iota == s, incl - nvs, 0))
    nv_tot = jnp.max(incl)

    def _pub(i, _):
        pltpu.sync_copy(candk.at[pl.ds(i * L, L)],
                        sh_gk.at[pl.ds((my_slot + i) * L, L)])
        pltpu.sync_copy(candi.at[pl.ds(i * L, L)],
                        sh_gi.at[pl.ds((my_slot + i) * L, L)])
        return 0
    lax.fori_loop(0, nv, _pub, 0)
    plsc.subcore_barrier()

    nb = (nv_tot + 15) // 16

    def _pull(i, _):
        pltpu.sync_copy(sh_gk.at[pl.ds(i * 256, 256)], gk.at[pl.ds(i * 256, 256)])
        pltpu.sync_copy(sh_gi.at[pl.ds(i * 256, 256)], gi.at[pl.ds(i * 256, 256)])
        return 0
    lax.fori_loop(0, nb, _pull, 0)

    def _zero_sel(i, _):
        selbuf[pl.ds(i * L, L)] = zeros
        return 0
    lax.fori_loop(0, L, _zero_sel, 0)

    n_el = nv_tot * L
    n_mine = jnp.maximum(0, (nv_tot - s + 15) // 16)

    def _rank_one(i, _):
        v = s + i * 16
        mk = gk[pl.ds(v * L, L)]
        mi = gi[pl.ds(v * L, L)]

        def _cmp(j, rank):
            jv = jnp.full((L,), j, jnp.int32)
            kj = plsc.load_gather(gk, [jv])
            ij = plsc.load_gather(gi, [jv])
            beats = jnp.logical_or(kj > mk,
                                   jnp.logical_and(kj == mk, ij < mi))
            return rank + beats.astype(jnp.int32)
        rank = lax.fori_loop(0, n_el, _cmp, zeros)
        plsc.store_scatter(selbuf, [rank], mi, mask=rank < K)
        return 0
    lax.fori_loop(0, n_mine, _rank_one, 0)

    pltpu.sync_copy(selbuf, sh_sel.at[s])
    plsc.subcore_barrier()

    for t in range(NUM_TILES):
        pltpu.sync_copy(sh_sel.at[t, pl.ds(s * L, L)], selrows.at[t])
    acc = selrows[0, :]
    for t in range(1, NUM_TILES):
        acc = acc + selrows[t, :]
    selidx[...] = jnp.minimum(acc, N - 1)

    @pl.when(c == 0)
    def _():
        pltpu.async_copy(src_hbm.at[selidx], gsrc, sem).wait()
        pltpu.async_copy(tgt_hbm.at[selidx], gtgt, sem).wait()
        pltpu.async_copy(ovl_hbm.at[selidx], govl, sem).wait()
        pltpu.sync_copy(gsrc, out_src.at[pl.ds(s * L, L)])
        pltpu.sync_copy(gtgt, out_tgt.at[pl.ds(s * L, L)])
        pltpu.sync_copy(govl, out_ovl.at[pl.ds(s * L, L)])


@functools.lru_cache(maxsize=1)
def _build():
    mesh = plsc.VectorSubcoreMesh(core_axis_name="c", subcore_axis_name="s")
    return pl.kernel(
        _body,
        out_type=(jax.ShapeDtypeStruct((K,), jnp.int32),
                  jax.ShapeDtypeStruct((K,), jnp.int32),
                  jax.ShapeDtypeStruct((K,), jnp.float32)),
        mesh=mesh,
        scratch_types=[
            pltpu.VMEM((PER_TILE,), jnp.int32),
            pltpu.VMEM((PER_TILE,), jnp.int32),
            pltpu.VMEM((4096,), jnp.int32),
            pltpu.VMEM((256,), jnp.int32),
            pltpu.VMEM((CAP + L,), jnp.int32),
            pltpu.VMEM((CAP + L,), jnp.int32),
            pltpu.VMEM((CAP,), jnp.int32),
            pltpu.VMEM((CAP,), jnp.int32),
            pltpu.VMEM((NUM_TILES, 256), jnp.int32),
            pltpu.VMEM((256,), jnp.int32),
            pltpu.VMEM((NUM_TILES, L), jnp.int32),
            pltpu.VMEM((L,), jnp.int32),
            pltpu.VMEM((L,), jnp.int32),
            pltpu.VMEM((NUM_TILES, 8), jnp.int32),
            pltpu.VMEM((L,), jnp.int32),
            pltpu.VMEM((L,), jnp.int32),
            pltpu.VMEM((L,), jnp.float32),
            pltpu.VMEM_SHARED((NUM_TILES, 256), jnp.int32),
            pltpu.VMEM_SHARED((NUM_TILES, 8), jnp.int32),
            pltpu.VMEM_SHARED((CAP,), jnp.int32),
            pltpu.VMEM_SHARED((CAP,), jnp.int32),
            pltpu.VMEM_SHARED((NUM_TILES, 256), jnp.int32),
            pltpu.SemaphoreType.DMA,
        ],
        compiler_params=pltpu.CompilerParams(needs_layout_passes=False),
    )


def kernel(gt_src_corr_indices, gt_tgt_corr_indices, gt_corr_overlaps):
    n = gt_corr_overlaps.shape[0]
    scores = gt_corr_overlaps / jnp.sum(gt_corr_overlaps)
    gumbel = jax.random.gumbel(jax.random.key(1234), (n,),
                               dtype=gt_corr_overlaps.dtype)
    perturbed = jnp.log(scores) + gumbel
    pert_pad = jnp.concatenate(
        [perturbed, jnp.full((PAD_N - n,), -jnp.inf, jnp.float32)])
    pert_bits = jax.lax.bitcast_convert_type(pert_pad, jnp.int32)
    out_src, out_tgt, out_ovl = _build()(
        pert_bits, gt_src_corr_indices, gt_tgt_corr_indices, gt_corr_overlaps)
    return (out_src, out_tgt, out_ovl)
